# baseline (speedup 1.0000x reference)
import functools

import jax
import jax.numpy as jnp
from jax import lax
from jax.experimental import pallas as pl
from jax.experimental.pallas import tpu as pltpu
from jax.experimental.pallas import tpu_sc as plsc

N = 10000
E = 320000
D = 128
H = 128
L = 128
G = 64
NLAYERS = 3

NC = 2
NS = 16
NW = NC * NS

C = 80
NB = 2
NI = 8
CPT = 128
EWP = CPT * C
EP = NW * EWP

NPAD = 10112
RPT = NPAD // NS


@functools.cache
def _make_seg_sum():
  mesh = plsc.VectorSubcoreMesh(core_axis_name="c", subcore_axis_name="s",
                                num_cores=NC, num_subcores=NS)

  @functools.partial(
      pl.kernel,
      out_type=jax.ShapeDtypeStruct((NC * NPAD, D), jnp.float32),
      mesh=mesh,
      scratch_types=[
          pltpu.VMEM((NI, 1, C), jnp.int32),
          pltpu.VMEM((NI, 1, C), jnp.int32),
          pltpu.VMEM((NB, C, D), jnp.float32),
          pltpu.VMEM_SHARED((NPAD, D), jnp.float32),
          [pltpu.SemaphoreType.DMA] * NI,
          [pltpu.SemaphoreType.DMA] * NB,
          [pltpu.SemaphoreType.DMA] * NB,
      ],
  )
  def seg_sum(h_hbm, src_hbm, dst_hbm, zeros_hbm, out_hbm,
              srcb, dstb, rows_v, acc_sh, si, sg, ss):
    c = lax.axis_index("c")
    s = lax.axis_index("s")
    wid = c * NS + s
    r0 = s * RPT

    pltpu.sync_copy(zeros_hbm.at[pl.ds(r0, RPT)], acc_sh.at[pl.ds(r0, RPT)])
    plsc.subcore_barrier()

    def idx_load(chunk, bi):
      j = wid * CPT + chunk
      pltpu.async_copy(src_hbm.at[j], srcb.at[bi], si[bi])
      pltpu.async_copy(dst_hbm.at[j], dstb.at[bi], si[bi])

    def wait_idx(bi):
      for _ in range(2):
        pltpu.make_async_copy(src_hbm.at[0], srcb.at[bi], si[bi]).wait()

    def gather(bi, b):
      return pltpu.async_copy(h_hbm.at[srcb.at[bi, 0]], rows_v.at[b], sg[b])

    def wait_gather(b):
      pltpu.make_async_copy(h_hbm.at[srcb.at[0, 0]], rows_v.at[b],
                            sg[b]).wait()

    def scatter(bi, b):
      return pltpu.async_copy(rows_v.at[b], acc_sh.at[dstb.at[bi, 0]],
                              ss[b], add=True)

    for bi in range(NI):
      idx_load(bi, bi)
    for b in range(NB):
      wait_idx(b)
      gather(b, b)

    def body(k, carry):
      for u in range(NI):
        chunk = NI * k + u
        b = u % NB
        bi = u
        wait_gather(b)
        scatter(bi, b).wait()
        wait_idx((u + NB) % NI)
        gather((u + NB) % NI, b)
        idx_load(lax.min(chunk + NI, CPT - 1), bi)
      return carry

    lax.fori_loop(0, CPT // NI, body, 0)
    for b in range(NB):
      wait_gather(b)
    for bi in range(NB, NI):
      wait_idx(bi)
    plsc.subcore_barrier()
    pltpu.sync_copy(acc_sh.at[pl.ds(r0, RPT)],
                    out_hbm.at[pl.ds(c * NPAD + r0, RPT)])

  return seg_sum


def _leaky(x):
  return jnp.where(x >= 0, x, 0.2 * x)


def _layer_body(parts_ref, h_ref, w1_ref, b1_ref, g_ref, be_ref, w2_ref,
                b2_ref, o_ref):
  agg = parts_ref[0:N, :] + parts_ref[NPAD:NPAD + N, :]
  x = agg + h_ref[...]
  t = jnp.dot(x, w1_ref[...], preferred_element_type=jnp.float32)
  t = _leaky(t + b1_ref[...])
  mu = jnp.mean(t, axis=0, keepdims=True)
  var = jnp.mean((t - mu) * (t - mu), axis=0, keepdims=True)
  t = (t - mu) * lax.rsqrt(var + 1e-5) * g_ref[...] + be_ref[...]
  t = jnp.dot(t, w2_ref[...], preferred_element_type=jnp.float32)
  o_ref[...] = _leaky(t + b2_ref[...])


_layer_tc = pl.pallas_call(
    _layer_body,
    out_shape=jax.ShapeDtypeStruct((N, H), jnp.float32),
)


def _final_body(h_ref, batch_ref, wa_ref, ba_ref, gg_ref, beg_ref, wl_ref,
                bl_ref, o_ref):
  h = h_ref[...]
  s = jnp.sum(h * wa_ref[...], axis=1, keepdims=True) + ba_ref[...]
  att = jax.nn.sigmoid(s)
  hw = h * att
  gid = lax.broadcasted_iota(jnp.int32, (G, N), 0)
  m = (gid == batch_ref[...]).astype(jnp.float32)
  pooled = jnp.dot(m, hw, preferred_element_type=jnp.float32)
  mu = jnp.mean(pooled, axis=0, keepdims=True)
  var = jnp.mean((pooled - mu) * (pooled - mu), axis=0, keepdims=True)
  nrm = (pooled - mu) * lax.rsqrt(var + 1e-5) * gg_ref[...] + beg_ref[...]
  o_ref[...] = jnp.dot(nrm, wl_ref[...],
                       preferred_element_type=jnp.float32) + bl_ref[...]


_final_tc = pl.pallas_call(
    _final_body,
    out_shape=jax.ShapeDtypeStruct((G, L), jnp.float32),
)


def kernel(x, edge_index, batch, params):
  src = edge_index[0]
  dst = edge_index[1]
  pad = EP - E
  pad_src = (jnp.arange(pad, dtype=jnp.int32) * 8) % N
  pad_dst = N + (jnp.arange(pad, dtype=jnp.int32) % (NPAD - N))
  src_p3 = jnp.concatenate([src, pad_src]).reshape(NW * CPT, 1, C)
  dst_p3 = jnp.concatenate([dst, pad_dst]).reshape(NW * CPT, 1, C)
  zeros = jnp.zeros((NPAD, D), jnp.float32)
  batch2d = batch.reshape(1, N)

  h = x
  for i in range(NLAYERS):
    parts = _make_seg_sum()(h, src_p3, dst_p3, zeros)
    h = _layer_tc(
        parts, h,
        params['W1_%d' % i], params['b1_%d' % i].reshape(1, H),
        params['g_%d' % i].reshape(1, H), params['be_%d' % i].reshape(1, H),
        params['W2_%d' % i], params['b2_%d' % i].reshape(1, H))

  out = _final_tc(
      h, batch2d, params['Wa'].reshape(1, H), params['ba'].reshape(1, 1),
      params['g_glob'].reshape(1, H), params['be_glob'].reshape(1, H),
      params['Wl'], params['bl'].reshape(1, L))
  return (h, out)


# speedup vs baseline: 1.1874x; 1.1874x over previous
import functools

import jax
import jax.numpy as jnp
from jax import lax
from jax.experimental import pallas as pl
from jax.experimental.pallas import tpu as pltpu
from jax.experimental.pallas import tpu_sc as plsc

N = 10000
E = 320000
D = 128
H = 128
L = 128
G = 64
NLAYERS = 3

NC = 2
NS = 16
NW = NC * NS

C = 80
NB = 4
NI = 8
CPT = 128
EWP = CPT * C
EP = NW * EWP

NPAD = 10112
RPT = NPAD // NS


@functools.cache
def _make_seg_sum():
  mesh = plsc.VectorSubcoreMesh(core_axis_name="c", subcore_axis_name="s",
                                num_cores=NC, num_subcores=NS)

  @functools.partial(
      pl.kernel,
      out_type=jax.ShapeDtypeStruct((NC * NPAD, D), jnp.float32),
      mesh=mesh,
      scratch_types=[
          pltpu.VMEM((NI, 1, C), jnp.int32),
          pltpu.VMEM((NI, 1, C), jnp.int32),
          pltpu.VMEM((NB, C, D), jnp.float32),
          pltpu.VMEM_SHARED((NPAD, D), jnp.float32),
          [pltpu.SemaphoreType.DMA] * NI,
          [pltpu.SemaphoreType.DMA] * NB,
          [pltpu.SemaphoreType.DMA] * NB,
      ],
  )
  def seg_sum(h_hbm, src_hbm, dst_hbm, zeros_hbm, out_hbm,
              srcb, dstb, rows_v, acc_sh, si, sg, ss):
    c = lax.axis_index("c")
    s = lax.axis_index("s")
    wid = c * NS + s
    r0 = s * RPT

    pltpu.sync_copy(zeros_hbm.at[pl.ds(r0, RPT)], acc_sh.at[pl.ds(r0, RPT)])
    plsc.subcore_barrier()

    def idx_load(chunk, bi):
      j = wid * CPT + chunk
      pltpu.async_copy(src_hbm.at[j], srcb.at[bi], si[bi])
      pltpu.async_copy(dst_hbm.at[j], dstb.at[bi], si[bi])

    def wait_idx(bi):
      for _ in range(2):
        pltpu.make_async_copy(src_hbm.at[0], srcb.at[bi], si[bi]).wait()

    def gather(bi, b):
      return pltpu.async_copy(h_hbm.at[srcb.at[bi, 0]], rows_v.at[b], sg[b])

    def wait_gather(b):
      pltpu.make_async_copy(h_hbm.at[srcb.at[0, 0]], rows_v.at[b],
                            sg[b]).wait()

    def scatter(bi, b):
      return pltpu.async_copy(rows_v.at[b], acc_sh.at[dstb.at[bi, 0]],
                              ss[b], add=True)

    for bi in range(NI):
      idx_load(bi, bi)
    for b in range(NB):
      wait_idx(b)
      gather(b, b)

    def body(k, carry):
      for u in range(NI):
        chunk = NI * k + u
        b = u % NB
        bi = u
        wait_gather(b)
        scatter(bi, b).wait()
        wait_idx((u + NB) % NI)
        gather((u + NB) % NI, b)
        idx_load(lax.min(chunk + NI, CPT - 1), bi)
      return carry

    lax.fori_loop(0, CPT // NI, body, 0)
    for b in range(NB):
      wait_gather(b)
    for bi in range(NB, NI):
      wait_idx(bi)
    plsc.subcore_barrier()
    pltpu.sync_copy(acc_sh.at[pl.ds(r0, RPT)],
                    out_hbm.at[pl.ds(c * NPAD + r0, RPT)])

  return seg_sum


def _leaky(x):
  return jnp.where(x >= 0, x, 0.2 * x)


def _layer_body(parts_ref, h_ref, w1_ref, b1_ref, g_ref, be_ref, w2_ref,
                b2_ref, o_ref):
  agg = parts_ref[0:N, :] + parts_ref[NPAD:NPAD + N, :]
  x = agg + h_ref[...]
  t = jnp.dot(x, w1_ref[...], preferred_element_type=jnp.float32)
  t = _leaky(t + b1_ref[...])
  mu = jnp.mean(t, axis=0, keepdims=True)
  var = jnp.mean((t - mu) * (t - mu), axis=0, keepdims=True)
  t = (t - mu) * lax.rsqrt(var + 1e-5) * g_ref[...] + be_ref[...]
  t = jnp.dot(t, w2_ref[...], preferred_element_type=jnp.float32)
  o_ref[...] = _leaky(t + b2_ref[...])


_layer_tc = pl.pallas_call(
    _layer_body,
    out_shape=jax.ShapeDtypeStruct((N, H), jnp.float32),
)


def _layer_final_body(parts_ref, h_ref, w1_ref, b1_ref, g_ref, be_ref,
                      w2_ref, b2_ref, batch_ref, wa_ref, ba_ref, gg_ref,
                      beg_ref, wl_ref, bl_ref, o_ref, oo_ref):
  agg = parts_ref[0:N, :] + parts_ref[NPAD:NPAD + N, :]
  x = agg + h_ref[...]
  t = jnp.dot(x, w1_ref[...], preferred_element_type=jnp.float32)
  t = _leaky(t + b1_ref[...])
  mu = jnp.mean(t, axis=0, keepdims=True)
  var = jnp.mean((t - mu) * (t - mu), axis=0, keepdims=True)
  t = (t - mu) * lax.rsqrt(var + 1e-5) * g_ref[...] + be_ref[...]
  t = jnp.dot(t, w2_ref[...], preferred_element_type=jnp.float32)
  h = _leaky(t + b2_ref[...])
  o_ref[...] = h
  s = jnp.sum(h * wa_ref[...], axis=1, keepdims=True) + ba_ref[...]
  att = jax.nn.sigmoid(s)
  hw = h * att
  gid = lax.broadcasted_iota(jnp.int32, (G, N), 0)
  m = (gid == batch_ref[...]).astype(jnp.float32)
  pooled = jnp.dot(m, hw, preferred_element_type=jnp.float32)
  mu = jnp.mean(pooled, axis=0, keepdims=True)
  var = jnp.mean((pooled - mu) * (pooled - mu), axis=0, keepdims=True)
  nrm = (pooled - mu) * lax.rsqrt(var + 1e-5) * gg_ref[...] + beg_ref[...]
  oo_ref[...] = jnp.dot(nrm, wl_ref[...],
                        preferred_element_type=jnp.float32) + bl_ref[...]


_layer_final_tc = pl.pallas_call(
    _layer_final_body,
    out_shape=(
        jax.ShapeDtypeStruct((N, H), jnp.float32),
        jax.ShapeDtypeStruct((G, L), jnp.float32),
    ),
)


def kernel(x, edge_index, batch, params):
  src = edge_index[0]
  dst = edge_index[1]
  pad = EP - E
  pad_src = (jnp.arange(pad, dtype=jnp.int32) * 8) % N
  pad_dst = N + (jnp.arange(pad, dtype=jnp.int32) % (NPAD - N))
  src_p3 = jnp.concatenate([src, pad_src]).reshape(NW * CPT, 1, C)
  dst_p3 = jnp.concatenate([dst, pad_dst]).reshape(NW * CPT, 1, C)
  zeros = jnp.zeros((NPAD, D), jnp.float32)
  batch2d = batch.reshape(1, N)

  h = x
  for i in range(NLAYERS - 1):
    parts = _make_seg_sum()(h, src_p3, dst_p3, zeros)
    h = _layer_tc(
        parts, h,
        params['W1_%d' % i], params['b1_%d' % i].reshape(1, H),
        params['g_%d' % i].reshape(1, H), params['be_%d' % i].reshape(1, H),
        params['W2_%d' % i], params['b2_%d' % i].reshape(1, H))

  i = NLAYERS - 1
  parts = _make_seg_sum()(h, src_p3, dst_p3, zeros)
  h, out = _layer_final_tc(
      parts, h,
      params['W1_%d' % i], params['b1_%d' % i].reshape(1, H),
      params['g_%d' % i].reshape(1, H), params['be_%d' % i].reshape(1, H),
      params['W2_%d' % i], params['b2_%d' % i].reshape(1, H),
      batch2d, params['Wa'].reshape(1, H), params['ba'].reshape(1, 1),
      params['g_glob'].reshape(1, H), params['be_glob'].reshape(1, H),
      params['Wl'], params['bl'].reshape(1, L))
  return (h, out)


# speedup vs baseline: 1.1987x; 1.0095x over previous
import functools

import jax
import jax.numpy as jnp
from jax import lax
from jax.experimental import pallas as pl
from jax.experimental.pallas import tpu as pltpu
from jax.experimental.pallas import tpu_sc as plsc

N = 10000
E = 320000
D = 128
H = 128
L = 128
G = 64
NLAYERS = 3

NC = 2
NS = 16
NW = NC * NS

C = 56
NB = 6
NI = 12
CPT = 180
EWP = CPT * C
EP = NW * EWP

NPAD = 10112
RPT = NPAD // NS


@functools.cache
def _make_seg_sum():
  mesh = plsc.VectorSubcoreMesh(core_axis_name="c", subcore_axis_name="s",
                                num_cores=NC, num_subcores=NS)

  @functools.partial(
      pl.kernel,
      out_type=jax.ShapeDtypeStruct((NC * NPAD, D), jnp.float32),
      mesh=mesh,
      scratch_types=[
          pltpu.VMEM((NI, 1, C), jnp.int32),
          pltpu.VMEM((NI, 1, C), jnp.int32),
          pltpu.VMEM((NB, C, D), jnp.float32),
          pltpu.VMEM_SHARED((NPAD, D), jnp.float32),
          [pltpu.SemaphoreType.DMA] * NI,
          [pltpu.SemaphoreType.DMA] * NB,
          [pltpu.SemaphoreType.DMA] * NB,
      ],
  )
  def seg_sum(h_hbm, src_hbm, dst_hbm, zeros_hbm, out_hbm,
              srcb, dstb, rows_v, acc_sh, si, sg, ss):
    c = lax.axis_index("c")
    s = lax.axis_index("s")
    wid = c * NS + s
    r0 = s * RPT

    pltpu.sync_copy(zeros_hbm.at[pl.ds(r0, RPT)], acc_sh.at[pl.ds(r0, RPT)])
    plsc.subcore_barrier()

    def idx_load(chunk, bi):
      j = wid * CPT + chunk
      pltpu.async_copy(src_hbm.at[j], srcb.at[bi], si[bi])
      pltpu.async_copy(dst_hbm.at[j], dstb.at[bi], si[bi])

    def wait_idx(bi):
      for _ in range(2):
        pltpu.make_async_copy(src_hbm.at[0], srcb.at[bi], si[bi]).wait()

    def gather(bi, b):
      return pltpu.async_copy(h_hbm.at[srcb.at[bi, 0]], rows_v.at[b], sg[b])

    def wait_gather(b):
      pltpu.make_async_copy(h_hbm.at[srcb.at[0, 0]], rows_v.at[b],
                            sg[b]).wait()

    def scatter(bi, b):
      return pltpu.async_copy(rows_v.at[b], acc_sh.at[dstb.at[bi, 0]],
                              ss[b], add=True)

    for bi in range(NI):
      idx_load(bi, bi)
    for b in range(NB):
      wait_idx(b)
      gather(b, b)

    def body(k, carry):
      for u in range(NI):
        chunk = NI * k + u
        b = u % NB
        bi = u
        wait_gather(b)
        scatter(bi, b).wait()
        wait_idx((u + NB) % NI)
        gather((u + NB) % NI, b)
        idx_load(lax.min(chunk + NI, CPT - 1), bi)
      return carry

    lax.fori_loop(0, CPT // NI, body, 0)
    for b in range(NB):
      wait_gather(b)
    for bi in range(NB, NI):
      wait_idx(bi)
    plsc.subcore_barrier()
    pltpu.sync_copy(acc_sh.at[pl.ds(r0, RPT)],
                    out_hbm.at[pl.ds(c * NPAD + r0, RPT)])

  return seg_sum


def _leaky(x):
  return jnp.where(x >= 0, x, 0.2 * x)


def _layer_body(parts_ref, h_ref, w1_ref, b1_ref, g_ref, be_ref, w2_ref,
                b2_ref, o_ref):
  agg = parts_ref[0:N, :] + parts_ref[NPAD:NPAD + N, :]
  x = agg + h_ref[...]
  t = jnp.dot(x, w1_ref[...], preferred_element_type=jnp.float32)
  t = _leaky(t + b1_ref[...])
  mu = jnp.mean(t, axis=0, keepdims=True)
  var = jnp.mean((t - mu) * (t - mu), axis=0, keepdims=True)
  t = (t - mu) * lax.rsqrt(var + 1e-5) * g_ref[...] + be_ref[...]
  t = jnp.dot(t, w2_ref[...], preferred_element_type=jnp.float32)
  o_ref[...] = _leaky(t + b2_ref[...])


_layer_tc = pl.pallas_call(
    _layer_body,
    out_shape=jax.ShapeDtypeStruct((N, H), jnp.float32),
)


def _final_body(h_ref, batch_ref, wa_ref, ba_ref, gg_ref, beg_ref, wl_ref,
                bl_ref, o_ref):
  h = h_ref[...]
  s = jnp.sum(h * wa_ref[...], axis=1, keepdims=True) + ba_ref[...]
  att = jax.nn.sigmoid(s)
  hw = h * att
  gid = lax.broadcasted_iota(jnp.int32, (G, N), 0)
  m = (gid == batch_ref[...]).astype(jnp.float32)
  pooled = jnp.dot(m, hw, preferred_element_type=jnp.float32)
  mu = jnp.mean(pooled, axis=0, keepdims=True)
  var = jnp.mean((pooled - mu) * (pooled - mu), axis=0, keepdims=True)
  nrm = (pooled - mu) * lax.rsqrt(var + 1e-5) * gg_ref[...] + beg_ref[...]
  o_ref[...] = jnp.dot(nrm, wl_ref[...],
                       preferred_element_type=jnp.float32) + bl_ref[...]


_final_tc = pl.pallas_call(
    _final_body,
    out_shape=jax.ShapeDtypeStruct((G, L), jnp.float32),
)


def kernel(x, edge_index, batch, params):
  src = edge_index[0]
  dst = edge_index[1]
  pad = EP - E
  pad_src = (jnp.arange(pad, dtype=jnp.int32) * 8) % N
  pad_dst = N + (jnp.arange(pad, dtype=jnp.int32) % (NPAD - N))
  src_p3 = jnp.concatenate([src, pad_src]).reshape(NW * CPT, 1, C)
  dst_p3 = jnp.concatenate([dst, pad_dst]).reshape(NW * CPT, 1, C)
  zeros = jnp.zeros((NPAD, D), jnp.float32)
  batch2d = batch.reshape(1, N)

  h = x
  for i in range(NLAYERS):
    parts = _make_seg_sum()(h, src_p3, dst_p3, zeros)
    h = _layer_tc(
        parts, h,
        params['W1_%d' % i], params['b1_%d' % i].reshape(1, H),
        params['g_%d' % i].reshape(1, H), params['be_%d' % i].reshape(1, H),
        params['W2_%d' % i], params['b2_%d' % i].reshape(1, H))

  out = _final_tc(
      h, batch2d, params['Wa'].reshape(1, H), params['ba'].reshape(1, 1),
      params['g_glob'].reshape(1, H), params['be_glob'].reshape(1, H),
      params['Wl'], params['bl'].reshape(1, L))
  return (h, out)


# speedup vs baseline: 1.2113x; 1.0105x over previous
import functools

import jax
import jax.numpy as jnp
from jax import lax
from jax.experimental import pallas as pl
from jax.experimental.pallas import tpu as pltpu
from jax.experimental.pallas import tpu_sc as plsc

N = 10000
E = 320000
D = 128
H = 128
L = 128
G = 64
NLAYERS = 3

NC = 2
NS = 16
NW = NC * NS

C = 64
NB = 4
NI = 8
CPT = 160
EWP = CPT * C
EP = NW * EWP

NPAD = 10112
RPT = NPAD // NS


@functools.cache
def _make_seg_sum():
  mesh = plsc.VectorSubcoreMesh(core_axis_name="c", subcore_axis_name="s",
                                num_cores=NC, num_subcores=NS)

  @functools.partial(
      pl.kernel,
      out_type=jax.ShapeDtypeStruct((NC * NPAD, D), jnp.float32),
      mesh=mesh,
      scratch_types=[
          pltpu.VMEM((NI, 1, C), jnp.int32),
          pltpu.VMEM((NI, 1, C), jnp.int32),
          pltpu.VMEM((NB, C, D), jnp.float32),
          pltpu.VMEM_SHARED((NPAD, D), jnp.float32),
          [pltpu.SemaphoreType.DMA] * NI,
          [pltpu.SemaphoreType.DMA] * NB,
          [pltpu.SemaphoreType.DMA] * NB,
      ],
  )
  def seg_sum(h_hbm, src_hbm, dst_hbm, zeros_hbm, out_hbm,
              srcb, dstb, rows_v, acc_sh, si, sg, ss):
    c = lax.axis_index("c")
    s = lax.axis_index("s")
    wid = c * NS + s
    r0 = s * RPT

    pltpu.sync_copy(zeros_hbm.at[pl.ds(r0, RPT)], acc_sh.at[pl.ds(r0, RPT)])
    plsc.subcore_barrier()

    def idx_load(chunk, bi):
      j = wid * CPT + chunk
      pltpu.async_copy(src_hbm.at[j], srcb.at[bi], si[bi])
      pltpu.async_copy(dst_hbm.at[j], dstb.at[bi], si[bi])

    def wait_idx(bi):
      for _ in range(2):
        pltpu.make_async_copy(src_hbm.at[0], srcb.at[bi], si[bi]).wait()

    def gather(bi, b):
      return pltpu.async_copy(h_hbm.at[srcb.at[bi, 0]], rows_v.at[b], sg[b])

    def wait_gather(b):
      pltpu.make_async_copy(h_hbm.at[srcb.at[0, 0]], rows_v.at[b],
                            sg[b]).wait()

    def scatter(bi, b):
      return pltpu.async_copy(rows_v.at[b], acc_sh.at[dstb.at[bi, 0]],
                              ss[b], add=True)

    for bi in range(NI):
      idx_load(bi, bi)
    for b in range(NB):
      wait_idx(b)
      gather(b, b)

    def body(k, carry):
      for u in range(NI):
        chunk = NI * k + u
        b = u % NB
        bi = u
        wait_gather(b)
        scatter(bi, b).wait()
        wait_idx((u + NB) % NI)
        gather((u + NB) % NI, b)
        idx_load(lax.min(chunk + NI, CPT - 1), bi)
      return carry

    lax.fori_loop(0, CPT // NI, body, 0)
    for b in range(NB):
      wait_gather(b)
    for bi in range(NB, NI):
      wait_idx(bi)
    plsc.subcore_barrier()
    pltpu.sync_copy(acc_sh.at[pl.ds(r0, RPT)],
                    out_hbm.at[pl.ds(c * NPAD + r0, RPT)])

  return seg_sum


def _leaky(x):
  return jnp.where(x >= 0, x, 0.2 * x)


def _layer_body(parts_ref, h_ref, w1_ref, b1_ref, g_ref, be_ref, w2_ref,
                b2_ref, o_ref):
  agg = parts_ref[0:N, :] + parts_ref[NPAD:NPAD + N, :]
  x = agg + h_ref[...]
  t = jnp.dot(x, w1_ref[...], preferred_element_type=jnp.float32)
  t = _leaky(t + b1_ref[...])
  mu = jnp.mean(t, axis=0, keepdims=True)
  var = jnp.mean((t - mu) * (t - mu), axis=0, keepdims=True)
  t = (t - mu) * lax.rsqrt(var + 1e-5) * g_ref[...] + be_ref[...]
  t = jnp.dot(t, w2_ref[...], preferred_element_type=jnp.float32)
  o_ref[...] = _leaky(t + b2_ref[...])


_layer_tc = pl.pallas_call(
    _layer_body,
    out_shape=jax.ShapeDtypeStruct((N, H), jnp.float32),
)


def _final_body(h_ref, batch_ref, wa_ref, ba_ref, gg_ref, beg_ref, wl_ref,
                bl_ref, o_ref):
  h = h_ref[...]
  s = jnp.sum(h * wa_ref[...], axis=1, keepdims=True) + ba_ref[...]
  att = jax.nn.sigmoid(s)
  hw = h * att
  gid = lax.broadcasted_iota(jnp.int32, (G, N), 0)
  m = (gid == batch_ref[...]).astype(jnp.float32)
  pooled = jnp.dot(m, hw, preferred_element_type=jnp.float32)
  mu = jnp.mean(pooled, axis=0, keepdims=True)
  var = jnp.mean((pooled - mu) * (pooled - mu), axis=0, keepdims=True)
  nrm = (pooled - mu) * lax.rsqrt(var + 1e-5) * gg_ref[...] + beg_ref[...]
  o_ref[...] = jnp.dot(nrm, wl_ref[...],
                       preferred_element_type=jnp.float32) + bl_ref[...]


_final_tc = pl.pallas_call(
    _final_body,
    out_shape=jax.ShapeDtypeStruct((G, L), jnp.float32),
)


def kernel(x, edge_index, batch, params):
  src = edge_index[0]
  dst = edge_index[1]
  pad = EP - E
  pad_src = (jnp.arange(pad, dtype=jnp.int32) * 8) % N
  pad_dst = N + (jnp.arange(pad, dtype=jnp.int32) % (NPAD - N))
  src_p3 = jnp.concatenate([src, pad_src]).reshape(NW * CPT, 1, C)
  dst_p3 = jnp.concatenate([dst, pad_dst]).reshape(NW * CPT, 1, C)
  zeros = jnp.zeros((NPAD, D), jnp.float32)
  batch2d = batch.reshape(1, N)

  h = x
  for i in range(NLAYERS):
    parts = _make_seg_sum()(h, src_p3, dst_p3, zeros)
    h = _layer_tc(
        parts, h,
        params['W1_%d' % i], params['b1_%d' % i].reshape(1, H),
        params['g_%d' % i].reshape(1, H), params['be_%d' % i].reshape(1, H),
        params['W2_%d' % i], params['b2_%d' % i].reshape(1, H))

  out = _final_tc(
      h, batch2d, params['Wa'].reshape(1, H), params['ba'].reshape(1, 1),
      params['g_glob'].reshape(1, H), params['be_glob'].reshape(1, H),
      params['Wl'], params['bl'].reshape(1, L))
  return (h, out)
